# baseline (speedup 1.0000x reference)
import functools

import jax
import jax.numpy as jnp
from jax.experimental import pallas as pl

GRID = 64
EPS = 1e-3
TAPS = (-1, 0, 1)


def _subm_kernel(xm_ref, x0_ref, xp_ref, m_ref, w_ref, b_ref, o_ref):
    d = pl.program_id(0)
    nd = pl.num_programs(0)
    _, H, W, C = x0_ref.shape
    Co = o_ref.shape[3]
    acc = jnp.zeros((H * W, Co), jnp.float32)
    for ref, dz in ((xm_ref, -1), (x0_ref, 0), (xp_ref, 1)):
        valid = jnp.logical_and(d + dz >= 0, d + dz < nd).astype(jnp.float32)
        p = ref[0] * valid
        pp = jnp.pad(p, ((1, 1), (1, 1), (0, 0)))
        for dy in TAPS:
            for dx in TAPS:
                sh = jax.lax.slice(pp, (1 + dy, 1 + dx, 0), (1 + dy + H, 1 + dx + W, C))
                acc = acc + jnp.dot(sh.reshape(H * W, C), w_ref[dz + 1, dy + 1, dx + 1],
                                    preferred_element_type=jnp.float32)
    y = jnp.maximum(acc + b_ref[0], 0.0).reshape(H, W, Co)
    o_ref[0] = y * m_ref[0][..., None]


def _stride_kernel(xm_ref, x0_ref, xp_ref, mm_ref, m0_ref, mp_ref, w_ref, b_ref,
                   o_ref, m2_ref):
    o = pl.program_id(0)
    _, H, W, C = x0_ref.shape
    Ho, Wo = H // 2, W // 2
    Co = o_ref.shape[3]
    acc = jnp.zeros((Ho * Wo, Co), jnp.float32)
    macc = jnp.zeros((Ho, Wo), jnp.float32)
    for ref, mref, dz in ((xm_ref, mm_ref, -1), (x0_ref, m0_ref, 0), (xp_ref, mp_ref, 1)):
        valid = (2 * o + dz >= 0).astype(jnp.float32)
        p = ref[0] * valid
        m = mref[0] * valid
        pp = jnp.pad(p, ((1, 1), (1, 1), (0, 0)))
        mp = jnp.pad(m, ((1, 1), (1, 1)))
        for dy in TAPS:
            for dx in TAPS:
                sh = jax.lax.slice(pp, (1 + dy, 1 + dx, 0), (1 + dy + H, 1 + dx + W, C))
                sh = sh.reshape(Ho, 2, Wo, 2, C)[:, 0, :, 0, :]
                msh = jax.lax.slice(mp, (1 + dy, 1 + dx), (1 + dy + H, 1 + dx + W))
                msh = msh.reshape(Ho, 2, Wo, 2)[:, 0, :, 0]
                acc = acc + jnp.dot(sh.reshape(Ho * Wo, C), w_ref[dz + 1, dy + 1, dx + 1],
                                    preferred_element_type=jnp.float32)
                macc = jnp.maximum(macc, msh)
    y = jnp.maximum(acc + b_ref[0], 0.0).reshape(Ho, Wo, Co)
    o_ref[0] = y * macc[..., None]
    m2_ref[0] = macc


def _subm_conv(x, mask, w, b):
    D, H, W, C = x.shape
    Co = w.shape[-1]
    plane = lambda off: pl.BlockSpec(
        (1, H, W, C), lambda d: (jnp.clip(d + off, 0, D - 1), 0, 0, 0))
    return pl.pallas_call(
        _subm_kernel,
        grid=(D,),
        in_specs=[
            plane(-1), plane(0), plane(1),
            pl.BlockSpec((1, H, W), lambda d: (d, 0, 0)),
            pl.BlockSpec((3, 3, 3, C, Co), lambda d: (0, 0, 0, 0, 0)),
            pl.BlockSpec((1, Co), lambda d: (0, 0)),
        ],
        out_specs=pl.BlockSpec((1, H, W, Co), lambda d: (d, 0, 0, 0)),
        out_shape=jax.ShapeDtypeStruct((D, H, W, Co), jnp.float32),
    )(x, x, x, mask, w, b)


def _stride_conv(x, mask, w, b):
    D, H, W, C = x.shape
    Do, Ho, Wo = D // 2, H // 2, W // 2
    Co = w.shape[-1]
    plane = lambda off: pl.BlockSpec(
        (1, H, W, C), lambda d: (jnp.clip(2 * d + off, 0, D - 1), 0, 0, 0))
    mplane = lambda off: pl.BlockSpec(
        (1, H, W), lambda d: (jnp.clip(2 * d + off, 0, D - 1), 0, 0))
    return pl.pallas_call(
        _stride_kernel,
        grid=(Do,),
        in_specs=[
            plane(-1), plane(0), plane(1),
            mplane(-1), mplane(0), mplane(1),
            pl.BlockSpec((3, 3, 3, C, Co), lambda d: (0, 0, 0, 0, 0)),
            pl.BlockSpec((1, Co), lambda d: (0, 0)),
        ],
        out_specs=[
            pl.BlockSpec((1, Ho, Wo, Co), lambda d: (d, 0, 0, 0)),
            pl.BlockSpec((1, Ho, Wo), lambda d: (d, 0, 0)),
        ],
        out_shape=[
            jax.ShapeDtypeStruct((Do, Ho, Wo, Co), jnp.float32),
            jax.ShapeDtypeStruct((Do, Ho, Wo), jnp.float32),
        ],
    )(x, x, x, mask, mask, mask, w, b)


def _fold(w, g):
    return w * (g / jnp.sqrt(1.0 + EPS))


def kernel(features, coords, w0a, g0a, b0a, w0b, g0b, b0b, wd0, gd0, bd0,
           w1a, g1a, b1a, w1b, g1b, b1b, wd1, gd1, bd1,
           w2a, g2a, b2a, w2b, g2b, b2b, w2c, g2c, b2c):
    ci = coords.astype(jnp.int32)
    x = jnp.zeros((GRID, GRID, GRID, 16), jnp.float32).at[ci[:, 0], ci[:, 1], ci[:, 2]].set(features)
    mask = jnp.zeros((GRID, GRID, GRID), jnp.float32).at[ci[:, 0], ci[:, 1], ci[:, 2]].set(1.0)

    b = lambda v: v.reshape(1, -1)
    x = _subm_conv(x, mask, _fold(w0a, g0a), b(b0a))
    x = _subm_conv(x, mask, _fold(w0b, g0b), b(b0b))
    x, mask2 = _stride_conv(x, mask, _fold(wd0, gd0), b(bd0))
    x = _subm_conv(x, mask2, _fold(w1a, g1a), b(b1a))
    x = _subm_conv(x, mask2, _fold(w1b, g1b), b(b1b))
    x, mask3 = _stride_conv(x, mask2, _fold(wd1, gd1), b(bd1))
    x = _subm_conv(x, mask3, _fold(w2a, g2a), b(b2a))
    x = _subm_conv(x, mask3, _fold(w2b, g2b), b(b2b))
    x = _subm_conv(x, mask3, _fold(w2c, g2c), b(b2c))
    return jnp.transpose(x[None], (0, 4, 1, 2, 3))


# speedup vs baseline: 1.6511x; 1.6511x over previous
import functools

import jax
import jax.numpy as jnp
from jax.experimental import pallas as pl

GRID = 64
EPS = 1e-3
TAPS = (-1, 0, 1)


def _lshift(a, k):
    if k == 0:
        return a
    n = a.shape[-1]
    nd = a.ndim
    zero = ((0, 0),) * (nd - 1)
    if k > 0:
        p = jnp.pad(a, zero + ((0, k),))
        return jax.lax.slice_in_dim(p, k, k + n, axis=nd - 1)
    p = jnp.pad(a, zero + ((-k, 0),))
    return jax.lax.slice_in_dim(p, 0, n, axis=nd - 1)


def _wrap_masks(W, N):
    w = jax.lax.broadcasted_iota(jnp.int32, (1, N), 1) % W
    mm = (w != 0).astype(jnp.float32)
    mp = (w != W - 1).astype(jnp.float32)
    return mm, mp


def _im2col(p, mm, mp):
    return jnp.concatenate([_lshift(p, -1) * mm, p, _lshift(p, 1) * mp], axis=0)


def _subm_t_kernel(W, xm_ref, x0_ref, xp_ref, m_ref, w_ref, b_ref, o_ref):
    d = pl.program_id(0)
    nd = pl.num_programs(0)
    _, Ci, N = x0_ref.shape
    _, Co, _ = o_ref.shape
    mm, mp = _wrap_masks(W, N)
    acc = jnp.zeros((Co, N), jnp.float32)
    for ref, dz in ((xm_ref, -1), (x0_ref, 0), (xp_ref, 1)):
        valid = jnp.logical_and(d + dz >= 0, d + dz < nd).astype(jnp.float32)
        im = _im2col(ref[0] * valid, mm, mp)
        ydz = jnp.dot(w_ref[dz + 1], im, preferred_element_type=jnp.float32)
        for dy in TAPS:
            blk = jax.lax.slice_in_dim(ydz, (dy + 1) * Co, (dy + 2) * Co, axis=0)
            acc = acc + _lshift(blk, dy * W)
    y = jnp.maximum(acc + b_ref[...], 0.0) * m_ref[0]
    o_ref[0] = y


def _stride_t_kernel(W, xm_ref, x0_ref, xp_ref, mm_ref, m0_ref, mp_ref,
                     w_ref, b_ref, o_ref, m2_ref):
    d = pl.program_id(0)
    _, Ci, N = x0_ref.shape
    _, Co, _ = o_ref.shape
    mm, mp = _wrap_masks(W, N)
    pz = jnp.zeros((1, N), jnp.float32)
    acc = jnp.zeros((Co, N), jnp.float32)
    for ref, mref, dz in ((xm_ref, mm_ref, -1), (x0_ref, m0_ref, 0),
                          (xp_ref, mp_ref, 1)):
        valid = (2 * d + dz >= 0).astype(jnp.float32)
        pz = jnp.maximum(pz, mref[0] * valid)
        im = _im2col(ref[0] * valid, mm, mp)
        ydz = jnp.dot(w_ref[dz + 1], im, preferred_element_type=jnp.float32)
        for dy in TAPS:
            blk = jax.lax.slice_in_dim(ydz, (dy + 1) * Co, (dy + 2) * Co, axis=0)
            acc = acc + _lshift(blk, dy * W)
    pooled = jnp.zeros((1, N), jnp.float32)
    for dx, wmask in ((-1, mm), (0, None), (1, mp)):
        q = _lshift(pz, dx)
        if wmask is not None:
            q = q * wmask
        for dy in TAPS:
            pooled = jnp.maximum(pooled, _lshift(q, dy * W))
    y = jnp.maximum(acc + b_ref[...], 0.0) * pooled
    o_ref[0] = y
    m2_ref[0] = pooled


def _subm_conv(x, mask, w, b):
    D, Ci, N = x.shape
    Co = w.shape[1] // 3
    W = int(round(N ** 0.5))
    plane = lambda off: pl.BlockSpec(
        (1, Ci, N), lambda d: (jnp.clip(d + off, 0, D - 1), 0, 0))
    return pl.pallas_call(
        functools.partial(_subm_t_kernel, W),
        grid=(D,),
        in_specs=[
            plane(-1), plane(0), plane(1),
            pl.BlockSpec((1, 1, N), lambda d: (d, 0, 0)),
            pl.BlockSpec(w.shape, lambda d: (0, 0, 0)),
            pl.BlockSpec((Co, 1), lambda d: (0, 0)),
        ],
        out_specs=pl.BlockSpec((1, Co, N), lambda d: (d, 0, 0)),
        out_shape=jax.ShapeDtypeStruct((D, Co, N), jnp.float32),
    )(x, x, x, mask, w, b)


def _stride_conv(x, mask, w, b):
    D, Ci, N = x.shape
    Do = D // 2
    Co = w.shape[1] // 3
    W = int(round(N ** 0.5))
    plane = lambda off: pl.BlockSpec(
        (1, Ci, N), lambda d: (jnp.clip(2 * d + off, 0, D - 1), 0, 0))
    mplane = lambda off: pl.BlockSpec(
        (1, 1, N), lambda d: (jnp.clip(2 * d + off, 0, D - 1), 0, 0))
    y_full, m_full = pl.pallas_call(
        functools.partial(_stride_t_kernel, W),
        grid=(Do,),
        in_specs=[
            plane(-1), plane(0), plane(1),
            mplane(-1), mplane(0), mplane(1),
            pl.BlockSpec(w.shape, lambda d: (0, 0, 0)),
            pl.BlockSpec((Co, 1), lambda d: (0, 0)),
        ],
        out_specs=[
            pl.BlockSpec((1, Co, N), lambda d: (d, 0, 0)),
            pl.BlockSpec((1, 1, N), lambda d: (d, 0, 0)),
        ],
        out_shape=[
            jax.ShapeDtypeStruct((Do, Co, N), jnp.float32),
            jax.ShapeDtypeStruct((Do, 1, N), jnp.float32),
        ],
    )(x, x, x, mask, mask, mask, w, b)
    y = y_full.reshape(Do, Co, W, W)[:, :, ::2, ::2].reshape(Do, Co, N // 4)
    m2 = m_full.reshape(Do, W, W)[:, ::2, ::2].reshape(Do, 1, N // 4)
    return y, m2


def _fold(w, g):
    wf = w * (g / jnp.sqrt(1.0 + EPS))
    return jnp.transpose(wf, (0, 1, 4, 2, 3)).reshape(
        3, 3 * w.shape[4], 3 * w.shape[3])


def kernel(features, coords, w0a, g0a, b0a, w0b, g0b, b0b, wd0, gd0, bd0,
           w1a, g1a, b1a, w1b, g1b, b1b, wd1, gd1, bd1,
           w2a, g2a, b2a, w2b, g2b, b2b, w2c, g2c, b2c):
    ci = coords.astype(jnp.int32)
    x = jnp.zeros((GRID, GRID, GRID, 16), jnp.float32).at[ci[:, 0], ci[:, 1], ci[:, 2]].set(features)
    mask = jnp.zeros((GRID, GRID, GRID), jnp.float32).at[ci[:, 0], ci[:, 1], ci[:, 2]].set(1.0)
    x = jnp.transpose(x, (0, 3, 1, 2)).reshape(GRID, 16, GRID * GRID)
    mask = mask.reshape(GRID, 1, GRID * GRID)

    b = lambda v: v.reshape(-1, 1)
    x = _subm_conv(x, mask, _fold(w0a, g0a), b(b0a))
    x = _subm_conv(x, mask, _fold(w0b, g0b), b(b0b))
    x, mask2 = _stride_conv(x, mask, _fold(wd0, gd0), b(bd0))
    x = _subm_conv(x, mask2, _fold(w1a, g1a), b(b1a))
    x = _subm_conv(x, mask2, _fold(w1b, g1b), b(b1b))
    x, mask3 = _stride_conv(x, mask2, _fold(wd1, gd1), b(bd1))
    x = _subm_conv(x, mask3, _fold(w2a, g2a), b(b2a))
    x = _subm_conv(x, mask3, _fold(w2b, g2b), b(b2b))
    x = _subm_conv(x, mask3, _fold(w2c, g2c), b(b2c))
    D = x.shape[0]
    out = x.reshape(D, 64, D, D)
    return jnp.transpose(out, (1, 0, 2, 3))[None]


# speedup vs baseline: 1.6921x; 1.0248x over previous
import functools

import jax
import jax.numpy as jnp
from jax.experimental import pallas as pl

GRID = 64
EPS = 1e-3
TAPS = (-1, 0, 1)


def _lshift(a, k):
    if k == 0:
        return a
    n = a.shape[-1]
    nd = a.ndim
    zero = ((0, 0),) * (nd - 1)
    if k > 0:
        p = jnp.pad(a, zero + ((0, k),))
        return jax.lax.slice_in_dim(p, k, k + n, axis=nd - 1)
    p = jnp.pad(a, zero + ((-k, 0),))
    return jax.lax.slice_in_dim(p, 0, n, axis=nd - 1)


def _wrap_masks(W, N):
    w = jax.lax.broadcasted_iota(jnp.int32, (1, N), 1) % W
    mm = (w != 0).astype(jnp.float32)
    mp = (w != W - 1).astype(jnp.float32)
    return mm, mp


def _im2col(p, mm, mp):
    return jnp.concatenate([_lshift(p, -1) * mm, p, _lshift(p, 1) * mp], axis=0)


def _subm_t_kernel(W, xm_ref, x0_ref, xp_ref, m_ref, w_ref, b_ref, o_ref):
    d = pl.program_id(0)
    nd = pl.num_programs(0)
    _, Ci, N = x0_ref.shape
    _, Co, _ = o_ref.shape
    mm, mp = _wrap_masks(W, N)
    acc = jnp.zeros((Co, N), jnp.float32)
    for ref, dz in ((xm_ref, -1), (x0_ref, 0), (xp_ref, 1)):
        valid = jnp.logical_and(d + dz >= 0, d + dz < nd).astype(jnp.float32)
        im = _im2col(ref[0] * valid, mm, mp)
        ydz = jnp.dot(w_ref[dz + 1], im, preferred_element_type=jnp.float32)
        for dy in TAPS:
            blk = jax.lax.slice_in_dim(ydz, (dy + 1) * Co, (dy + 2) * Co, axis=0)
            acc = acc + _lshift(blk, dy * W)
    y = jnp.maximum(acc + b_ref[...], 0.0) * m_ref[0]
    o_ref[0] = y


def _stride_t_kernel(W, xm_ref, x0_ref, xp_ref, mm_ref, m0_ref, mp_ref,
                     w_ref, b_ref, o_ref, m2_ref):
    d = pl.program_id(0)
    _, Ci, N = x0_ref.shape
    _, Co, _ = o_ref.shape
    mm, mp = _wrap_masks(W, N)
    pz = jnp.zeros((1, N), jnp.float32)
    acc = jnp.zeros((Co, N), jnp.float32)
    for ref, mref, dz in ((xm_ref, mm_ref, -1), (x0_ref, m0_ref, 0),
                          (xp_ref, mp_ref, 1)):
        valid = (2 * d + dz >= 0).astype(jnp.float32)
        pz = jnp.maximum(pz, mref[0] * valid)
        im = _im2col(ref[0] * valid, mm, mp)
        ydz = jnp.dot(w_ref[dz + 1], im, preferred_element_type=jnp.float32)
        for dy in TAPS:
            blk = jax.lax.slice_in_dim(ydz, (dy + 1) * Co, (dy + 2) * Co, axis=0)
            acc = acc + _lshift(blk, dy * W)
    pooled = jnp.zeros((1, N), jnp.float32)
    for dx, wmask in ((-1, mm), (0, None), (1, mp)):
        q = _lshift(pz, dx)
        if wmask is not None:
            q = q * wmask
        for dy in TAPS:
            pooled = jnp.maximum(pooled, _lshift(q, dy * W))
    y = jnp.maximum(acc + b_ref[...], 0.0) * pooled
    o_ref[0] = y
    m2_ref[0] = pooled


def _subm_conv(x, mask, w, b):
    D, Ci, N = x.shape
    Co = w.shape[1] // 3
    W = int(round(N ** 0.5))
    plane = lambda off: pl.BlockSpec(
        (1, Ci, N), lambda d: (jnp.clip(d + off, 0, D - 1), 0, 0))
    return pl.pallas_call(
        functools.partial(_subm_t_kernel, W),
        grid=(D,),
        in_specs=[
            plane(-1), plane(0), plane(1),
            pl.BlockSpec((1, 1, N), lambda d: (d, 0, 0)),
            pl.BlockSpec(w.shape, lambda d: (0, 0, 0)),
            pl.BlockSpec((Co, 1), lambda d: (0, 0)),
        ],
        out_specs=pl.BlockSpec((1, Co, N), lambda d: (d, 0, 0)),
        out_shape=jax.ShapeDtypeStruct((D, Co, N), jnp.float32),
    )(x, x, x, mask, w, b)


def _stride_conv(x, mask, w, b):
    D, Ci, N = x.shape
    Do = D // 2
    Co = w.shape[1] // 3
    W = int(round(N ** 0.5))
    plane = lambda off: pl.BlockSpec(
        (1, Ci, N), lambda d: (jnp.clip(2 * d + off, 0, D - 1), 0, 0))
    mplane = lambda off: pl.BlockSpec(
        (1, 1, N), lambda d: (jnp.clip(2 * d + off, 0, D - 1), 0, 0))
    y_full, m_full = pl.pallas_call(
        functools.partial(_stride_t_kernel, W),
        grid=(Do,),
        in_specs=[
            plane(-1), plane(0), plane(1),
            mplane(-1), mplane(0), mplane(1),
            pl.BlockSpec(w.shape, lambda d: (0, 0, 0)),
            pl.BlockSpec((Co, 1), lambda d: (0, 0)),
        ],
        out_specs=[
            pl.BlockSpec((1, Co, N), lambda d: (d, 0, 0)),
            pl.BlockSpec((1, 1, N), lambda d: (d, 0, 0)),
        ],
        out_shape=[
            jax.ShapeDtypeStruct((Do, Co, N), jnp.float32),
            jax.ShapeDtypeStruct((Do, 1, N), jnp.float32),
        ],
    )(x, x, x, mask, mask, mask, w, b)
    y = y_full.reshape(Do, Co, W, W)[:, :, ::2, ::2].reshape(Do, Co, N // 4)
    m2 = m_full.reshape(Do, W, W)[:, ::2, ::2].reshape(Do, 1, N // 4)
    return y, m2


def _fold(w, g):
    wf = w * (g / jnp.sqrt(1.0 + EPS))
    return jnp.transpose(wf, (0, 1, 4, 2, 3)).reshape(
        3, 3 * w.shape[4], 3 * w.shape[3])


def kernel(features, coords, w0a, g0a, b0a, w0b, g0b, b0b, wd0, gd0, bd0,
           w1a, g1a, b1a, w1b, g1b, b1b, wd1, gd1, bd1,
           w2a, g2a, b2a, w2b, g2b, b2b, w2c, g2c, b2c):
    ci = coords.astype(jnp.int32)
    lin = ci[:, 1] * GRID + ci[:, 2]
    x = jnp.zeros((GRID, 16, GRID * GRID), jnp.float32).at[ci[:, 0], :, lin].set(features)
    mask = jnp.zeros((GRID, 1, GRID * GRID), jnp.float32).at[ci[:, 0], 0, lin].set(1.0)

    b = lambda v: v.reshape(-1, 1)
    x = _subm_conv(x, mask, _fold(w0a, g0a), b(b0a))
    x = _subm_conv(x, mask, _fold(w0b, g0b), b(b0b))
    x, mask2 = _stride_conv(x, mask, _fold(wd0, gd0), b(bd0))
    x = _subm_conv(x, mask2, _fold(w1a, g1a), b(b1a))
    x = _subm_conv(x, mask2, _fold(w1b, g1b), b(b1b))
    x, mask3 = _stride_conv(x, mask2, _fold(wd1, gd1), b(bd1))
    x = _subm_conv(x, mask3, _fold(w2a, g2a), b(b2a))
    x = _subm_conv(x, mask3, _fold(w2b, g2b), b(b2b))
    x = _subm_conv(x, mask3, _fold(w2c, g2c), b(b2c))
    D = x.shape[0]
    out = x.reshape(D, 64, D, D)
    return jnp.transpose(out, (1, 0, 2, 3))[None]


# speedup vs baseline: 2.1054x; 1.2443x over previous
import functools

import jax
import jax.numpy as jnp
from jax.experimental import pallas as pl

GRID = 64
EPS = 1e-3
TAPS = (-1, 0, 1)


def _lshift(a, k):
    if k == 0:
        return a
    n = a.shape[-1]
    nd = a.ndim
    zero = ((0, 0),) * (nd - 1)
    if k > 0:
        p = jnp.pad(a, zero + ((0, k),))
        return jax.lax.slice_in_dim(p, k, k + n, axis=nd - 1)
    p = jnp.pad(a, zero + ((-k, 0),))
    return jax.lax.slice_in_dim(p, 0, n, axis=nd - 1)


def _wrap_masks(W, N):
    w = jax.lax.broadcasted_iota(jnp.int32, (1, N), 1) % W
    mm = (w != 0).astype(jnp.float32)
    mp = (w != W - 1).astype(jnp.float32)
    return mm, mp


def _im2col(p, mm, mp):
    return jnp.concatenate([_lshift(p, -1) * mm, p, _lshift(p, 1) * mp], axis=0)


def _subm_t_kernel(W, xm_ref, x0_ref, xp_ref, m_ref, w_ref, b_ref, o_ref):
    d = pl.program_id(0)
    nd = pl.num_programs(0)
    _, Ci, N = x0_ref.shape
    _, Co, _ = o_ref.shape
    mm, mp = _wrap_masks(W, N)
    acc = jnp.zeros((Co, N), jnp.float32)
    for ref, dz in ((xm_ref, -1), (x0_ref, 0), (xp_ref, 1)):
        valid = jnp.logical_and(d + dz >= 0, d + dz < nd).astype(jnp.float32)
        im = _im2col(ref[0] * valid, mm, mp)
        ydz = jnp.dot(w_ref[dz + 1], im, preferred_element_type=jnp.float32)
        for dy in TAPS:
            blk = jax.lax.slice_in_dim(ydz, (dy + 1) * Co, (dy + 2) * Co, axis=0)
            acc = acc + _lshift(blk, dy * W)
    y = jnp.maximum(acc + b_ref[...], 0.0) * m_ref[0]
    o_ref[0] = y


def _stride_t_kernel(W, xm_ref, x0_ref, xp_ref, mm_ref, m0_ref, mp_ref,
                     w_ref, b_ref, o_ref, m2_ref):
    d = pl.program_id(0)
    _, Ci, N = x0_ref.shape
    _, Co, _ = o_ref.shape
    mm, mp = _wrap_masks(W, N)
    pz = jnp.zeros((1, N), jnp.float32)
    acc = jnp.zeros((Co, N), jnp.float32)
    for ref, mref, dz in ((xm_ref, mm_ref, -1), (x0_ref, m0_ref, 0),
                          (xp_ref, mp_ref, 1)):
        valid = (2 * d + dz >= 0).astype(jnp.float32)
        pz = jnp.maximum(pz, mref[0] * valid)
        im = _im2col(ref[0] * valid, mm, mp)
        ydz = jnp.dot(w_ref[dz + 1], im, preferred_element_type=jnp.float32)
        for dy in TAPS:
            blk = jax.lax.slice_in_dim(ydz, (dy + 1) * Co, (dy + 2) * Co, axis=0)
            acc = acc + _lshift(blk, dy * W)
    pooled = jnp.zeros((1, N), jnp.float32)
    for dx, wmask in ((-1, mm), (0, None), (1, mp)):
        q = _lshift(pz, dx)
        if wmask is not None:
            q = q * wmask
        for dy in TAPS:
            pooled = jnp.maximum(pooled, _lshift(q, dy * W))
    y = jnp.maximum(acc + b_ref[...], 0.0) * pooled
    o_ref[0] = y
    m2_ref[0] = pooled


def _subm_conv(x, mask, w, b):
    D, Ci, N = x.shape
    Co = w.shape[1] // 3
    W = int(round(N ** 0.5))
    plane = lambda off: pl.BlockSpec(
        (1, Ci, N), lambda d: (jnp.clip(d + off, 0, D - 1), 0, 0))
    return pl.pallas_call(
        functools.partial(_subm_t_kernel, W),
        grid=(D,),
        in_specs=[
            plane(-1), plane(0), plane(1),
            pl.BlockSpec((1, 1, N), lambda d: (d, 0, 0)),
            pl.BlockSpec(w.shape, lambda d: (0, 0, 0)),
            pl.BlockSpec((Co, 1), lambda d: (0, 0)),
        ],
        out_specs=pl.BlockSpec((1, Co, N), lambda d: (d, 0, 0)),
        out_shape=jax.ShapeDtypeStruct((D, Co, N), jnp.float32),
    )(x, x, x, mask, w, b)


def _stride_conv(x, mask, w, b):
    D, Ci, N = x.shape
    Do = D // 2
    Co = w.shape[1] // 3
    W = int(round(N ** 0.5))
    plane = lambda off: pl.BlockSpec(
        (1, Ci, N), lambda d: (jnp.clip(2 * d + off, 0, D - 1), 0, 0))
    mplane = lambda off: pl.BlockSpec(
        (1, 1, N), lambda d: (jnp.clip(2 * d + off, 0, D - 1), 0, 0))
    y_full, m_full = pl.pallas_call(
        functools.partial(_stride_t_kernel, W),
        grid=(Do,),
        in_specs=[
            plane(-1), plane(0), plane(1),
            mplane(-1), mplane(0), mplane(1),
            pl.BlockSpec(w.shape, lambda d: (0, 0, 0)),
            pl.BlockSpec((Co, 1), lambda d: (0, 0)),
        ],
        out_specs=[
            pl.BlockSpec((1, Co, N), lambda d: (d, 0, 0)),
            pl.BlockSpec((1, 1, N), lambda d: (d, 0, 0)),
        ],
        out_shape=[
            jax.ShapeDtypeStruct((Do, Co, N), jnp.float32),
            jax.ShapeDtypeStruct((Do, 1, N), jnp.float32),
        ],
    )(x, x, x, mask, mask, mask, w, b)
    y = y_full.reshape(Do, Co, W, W)[:, :, ::2, ::2].reshape(Do, Co, N // 4)
    m2 = m_full.reshape(Do, W, W)[:, ::2, ::2].reshape(Do, 1, N // 4)
    return y, m2


def _fold(w, g):
    wf = w * (g / jnp.sqrt(1.0 + EPS))
    return jnp.transpose(wf, (0, 1, 4, 2, 3)).reshape(
        3, 3 * w.shape[4], 3 * w.shape[3])


def kernel(features, coords, w0a, g0a, b0a, w0b, g0b, b0b, wd0, gd0, bd0,
           w1a, g1a, b1a, w1b, g1b, b1b, wd1, gd1, bd1,
           w2a, g2a, b2a, w2b, g2b, b2b, w2c, g2c, b2c):
    ci = coords.astype(jnp.int32)
    npts = features.shape[0]
    lin = (ci[:, 0] * GRID + ci[:, 1]) * GRID + ci[:, 2]
    slin, sidx = jax.lax.sort([lin, jnp.arange(npts, dtype=jnp.int32)],
                              num_keys=1, is_stable=True)
    winner = jnp.concatenate(
        [slin[:-1] != slin[1:], jnp.ones((1,), bool)])
    vals = features[sidx] * winner[:, None].astype(jnp.float32)
    x = jax.ops.segment_sum(vals, slin, num_segments=GRID * GRID * GRID)
    mask = jax.ops.segment_sum(winner.astype(jnp.float32), slin,
                               num_segments=GRID * GRID * GRID)
    x = jnp.transpose(x.reshape(GRID, GRID * GRID, 16), (0, 2, 1))
    x = x.reshape(GRID, 16, GRID * GRID)
    mask = mask.reshape(GRID, 1, GRID * GRID)

    b = lambda v: v.reshape(-1, 1)
    x = _subm_conv(x, mask, _fold(w0a, g0a), b(b0a))
    x = _subm_conv(x, mask, _fold(w0b, g0b), b(b0b))
    x, mask2 = _stride_conv(x, mask, _fold(wd0, gd0), b(bd0))
    x = _subm_conv(x, mask2, _fold(w1a, g1a), b(b1a))
    x = _subm_conv(x, mask2, _fold(w1b, g1b), b(b1b))
    x, mask3 = _stride_conv(x, mask2, _fold(wd1, gd1), b(bd1))
    x = _subm_conv(x, mask3, _fold(w2a, g2a), b(b2a))
    x = _subm_conv(x, mask3, _fold(w2b, g2b), b(b2b))
    x = _subm_conv(x, mask3, _fold(w2c, g2c), b(b2c))
    D = x.shape[0]
    out = x.reshape(D, 64, D, D)
    return jnp.transpose(out, (1, 0, 2, 3))[None]
